# Initial kernel scaffold; baseline (speedup 1.0000x reference)
import jax
import jax.numpy as jnp
from jax.experimental import pallas as pl


def kernel(x, edge_index, batch, link_indices, W1, b1, g1, be1, Wl, Wr, bs, g2, be2, W3, b3, g3, be3, W4, b4):
    raise NotImplementedError("write your pallas kernel here")


# speedup vs baseline: 6.8277x; 6.8277x over previous
import functools

import jax
import jax.numpy as jnp
from jax import lax
from jax.experimental import pallas as pl
from jax.experimental.pallas import tpu as pltpu
from jax.experimental.pallas import tpu_sc as plsc

N = 10000
E = 320000
G = 1024
NLINK = 2048
NC = 2
NS = 16
NW = NC * NS
CH = 128
BLK = 16
NCHUNK = 2560
E_PAD = NCHUNK * CH
N_ACC = 10112
POOL_R = 1152
N_POOL = 12288

_mesh = lambda: plsc.VectorSubcoreMesh(
    core_axis_name="c", subcore_axis_name="s", num_cores=NC, num_subcores=NS)


@functools.partial(
    pl.kernel,
    out_type=jax.ShapeDtypeStruct((NC, N_ACC, 16), jnp.float32),
    mesh=_mesh(),
    scratch_types=[
        pltpu.VMEM((NCHUNK // NW, CH), jnp.int32),
        pltpu.VMEM((CH, 16), jnp.float32),
        pltpu.VMEM((CH, 16), jnp.float32),
        pltpu.VMEM_SHARED((N_ACC, 16), jnp.float32),
    ],
)
def _sc_deg(dst_hbm, out_hbm, idst, onesb, zerob, spm):
    c = lax.axis_index("c")
    s = lax.axis_index("s")
    w = c * NS + s
    rpt = NCHUNK // NW
    zero16 = jnp.zeros((16,), jnp.float32)
    ones16 = jnp.ones((16,), jnp.float32)

    def fill(i, _):
        onesb[i, pl.ds(0, 16)] = ones16
        zerob[i, pl.ds(0, 16)] = zero16
        return 0
    lax.fori_loop(0, CH, fill, 0)
    pltpu.sync_copy(dst_hbm.at[pl.ds(w * rpt, rpt)], idst)
    wb = N_ACC // NS
    zb = s * wb
    for k in range(wb // CH):
        pltpu.sync_copy(zerob, spm.at[pl.ds(zb + k * CH, CH)])
    pltpu.sync_copy(zerob.at[pl.ds(0, wb % CH)],
                    spm.at[pl.ds(zb + (wb // CH) * CH, wb % CH)])
    plsc.subcore_barrier()

    def acc(j, _):
        pltpu.sync_copy(onesb, spm.at[idst.at[j]], add=True)
        return 0
    lax.fori_loop(0, rpt, acc, 0)
    plsc.subcore_barrier()
    pltpu.sync_copy(spm.at[pl.ds(s * wb, wb)], out_hbm.at[c, pl.ds(s * wb, wb)])


def _make_sc_agg(F, feature_split):
    ptc = NCHUNK // NS if feature_split else NCHUNK // NW
    nblk = ptc // BLK
    wb = N_ACC // NS

    @functools.partial(
        pl.kernel,
        out_type=jax.ShapeDtypeStruct((NC, N_ACC, F), jnp.float32),
        mesh=_mesh(),
        scratch_types=[
            pltpu.VMEM((BLK, CH), jnp.int32),
            pltpu.VMEM((BLK, CH), jnp.int32),
            pltpu.VMEM((CH, F), jnp.float32),
            pltpu.VMEM((CH, F), jnp.float32),
            pltpu.VMEM_SHARED((N_ACC, F), jnp.float32),
            pltpu.SemaphoreType.DMA,
            pltpu.SemaphoreType.DMA,
        ],
    )
    def agg(h_hbm, srca_hbm, srcb_hbm, dst_hbm, out_hbm,
            isrc, idst, buf0, buf1, acc, sem0, sem1):
        c = lax.axis_index("c")
        s = lax.axis_index("s")
        if feature_split:
            base = s * ptc
        else:
            base = (c * NS + s) * ptc
        zero16 = jnp.zeros((16,), jnp.float32)

        def zloop(i, _):
            buf0[i // (F // 16), pl.ds((i % (F // 16)) * 16, 16)] = zero16
            return 0
        lax.fori_loop(0, CH * F // 16, zloop, 0)
        zb = s * wb
        for k in range(wb // CH):
            pltpu.sync_copy(buf0, acc.at[pl.ds(zb + k * CH, CH)])
        pltpu.sync_copy(buf0.at[pl.ds(0, wb % CH)],
                        acc.at[pl.ds(zb + (wb // CH) * CH, wb % CH)])
        plsc.subcore_barrier()

        def blk(bi, _):
            bb = base + bi * BLK
            if feature_split:
                @pl.when(c == 0)
                def _():
                    pltpu.sync_copy(srca_hbm.at[pl.ds(bb, BLK)], isrc)

                @pl.when(c == 1)
                def _():
                    pltpu.sync_copy(srcb_hbm.at[pl.ds(bb, BLK)], isrc)
            else:
                pltpu.sync_copy(srca_hbm.at[pl.ds(bb, BLK)], isrc)
            pltpu.sync_copy(dst_hbm.at[pl.ds(bb, BLK)], idst)
            pltpu.async_copy(h_hbm.at[isrc.at[0]], buf0, sem0)
            pltpu.async_copy(h_hbm.at[isrc.at[1]], buf1, sem1)

            def step(k, _):
                @pl.when(k % 2 == 0)
                def _():
                    pltpu.make_async_copy(
                        h_hbm.at[isrc.at[k]], buf0, sem0).wait()
                    pltpu.sync_copy(buf0, acc.at[idst.at[k]], add=True)

                    @pl.when(k + 2 < BLK)
                    def _():
                        pltpu.async_copy(h_hbm.at[isrc.at[k + 2]], buf0, sem0)

                @pl.when(k % 2 == 1)
                def _():
                    pltpu.make_async_copy(
                        h_hbm.at[isrc.at[k]], buf1, sem1).wait()
                    pltpu.sync_copy(buf1, acc.at[idst.at[k]], add=True)

                    @pl.when(k + 2 < BLK)
                    def _():
                        pltpu.async_copy(h_hbm.at[isrc.at[k + 2]], buf1, sem1)
                return 0
            lax.fori_loop(0, BLK, step, 0)
            return 0
        lax.fori_loop(0, nblk, blk, 0)
        plsc.subcore_barrier()
        pltpu.sync_copy(acc.at[pl.ds(s * wb, wb)],
                        out_hbm.at[c, pl.ds(s * wb, wb)])

    return agg


_sc_agg256 = _make_sc_agg(128, True)
_sc_agg64 = _make_sc_agg(128, False)


@functools.partial(
    pl.kernel,
    out_type=jax.ShapeDtypeStruct((NC, POOL_R, 64), jnp.float32),
    mesh=_mesh(),
    scratch_types=[
        pltpu.VMEM((N_POOL // NW, 64), jnp.float32),
        pltpu.VMEM((8, CH), jnp.int32),
        pltpu.VMEM((CH, 64), jnp.float32),
        pltpu.VMEM_SHARED((POOL_R, 64), jnp.float32),
    ],
)
def _sc_pool(h_hbm, batch_hbm, out_hbm, rows, bidx, zerob, psh):
    c = lax.axis_index("c")
    s = lax.axis_index("s")
    w = c * NS + s
    npt = N_POOL // NW
    zero16 = jnp.zeros((16,), jnp.float32)

    def zloop(i, _):
        zerob[i >> 2, pl.ds((i & 3) * 16, 16)] = zero16
        return 0
    lax.fori_loop(0, CH * 4, zloop, 0)
    pltpu.sync_copy(h_hbm.at[pl.ds(w * npt, npt)], rows)
    pltpu.sync_copy(batch_hbm.at[pl.ds(w * 8, 8)], bidx)
    zr = POOL_R // NS
    pltpu.sync_copy(zerob.at[pl.ds(0, zr)], psh.at[pl.ds(s * zr, zr)])
    plsc.subcore_barrier()
    for k in range(npt // CH):
        pltpu.sync_copy(rows.at[pl.ds(k * CH, CH)], psh.at[bidx.at[k]],
                        add=True)
    plsc.subcore_barrier()
    pltpu.sync_copy(psh.at[pl.ds(s * zr, zr)], out_hbm.at[c, pl.ds(s * zr, zr)])


@functools.partial(
    pl.kernel,
    out_type=[jax.ShapeDtypeStruct((NLINK, 128), jnp.float32),
              jax.ShapeDtypeStruct((NLINK, 128), jnp.float32)],
    mesh=_mesh(),
    scratch_types=[
        pltpu.VMEM((64,), jnp.int32),
        pltpu.VMEM((64,), jnp.int32),
        pltpu.VMEM((64, 128), jnp.float32),
        pltpu.VMEM((64, 128), jnp.float32),
        pltpu.SemaphoreType.DMA,
    ],
)
def _sc_links(pool_hbm, li_hbm, ga_hbm, gb_hbm, ia, ib, ra, rb, sem):
    c = lax.axis_index("c")
    s = lax.axis_index("s")
    w = c * NS + s
    lpt = NLINK // NW
    pltpu.sync_copy(li_hbm.at[pl.ds(w * lpt, lpt)], ia)
    pltpu.sync_copy(li_hbm.at[pl.ds(NLINK + w * lpt, lpt)], ib)
    pltpu.async_copy(pool_hbm.at[ia], ra, sem).wait()
    pltpu.async_copy(pool_hbm.at[ib], rb, sem).wait()
    pltpu.sync_copy(ra, ga_hbm.at[pl.ds(w * lpt, lpt)])
    pltpu.sync_copy(rb, gb_hbm.at[pl.ds(w * lpt, lpt)])


def _bn_leaky(t, g, be):
    mu = jnp.mean(t, axis=0, keepdims=True)
    var = jnp.mean((t - mu) ** 2, axis=0, keepdims=True)
    bn = (t - mu) * lax.rsqrt(var + 1e-5) * g[None, :] + be[None, :]
    return jnp.where(bn >= 0, bn, 0.01 * bn)


def _tc_a(x_ref, w1_ref, deg_ref, y1_ref, dv_ref):
    deg = (deg_ref[0] + deg_ref[1])[:N, 0:1]
    dinv = lax.rsqrt(deg + 1.0)
    invc = 1.0 / jnp.maximum(deg, 1.0)
    xw = jnp.dot(x_ref[...], w1_ref[...], preferred_element_type=jnp.float32)
    y = xw * dinv
    y1_ref[0, :N, :] = y[:, :128]
    y1_ref[1, :N, :] = y[:, 128:]
    y1_ref[0, N:, :] = jnp.zeros((N_ACC - N, 128), jnp.float32)
    y1_ref[1, N:, :] = jnp.zeros((N_ACC - N, 128), jnp.float32)
    dv_ref[:N, :] = jnp.concatenate([dinv, invc], axis=1)
    dv_ref[N:, :] = jnp.zeros((N_ACC - N, 2), jnp.float32)


def _tc_b(s1_ref, y1_ref, dv_ref, b1_ref, g1_ref, be1_ref, h1_ref):
    dinv = dv_ref[:N, 0:1]
    for h in (0, 1):
        sl = slice(h * 128, (h + 1) * 128)
        t = dinv * (s1_ref[h, :N, :] + y1_ref[h, :N, :]) + b1_ref[sl][None, :]
        h1_ref[h, :N, :] = _bn_leaky(t, g1_ref[sl], be1_ref[sl])
        h1_ref[h, N:, :] = jnp.zeros((N_ACC - N, 128), jnp.float32)


def _tc_c(s2_ref, h1_ref, dv_ref, wl_ref, wr_ref, bs_ref, g2_ref, be2_ref,
          w3_ref, y3_ref):
    invc = dv_ref[:N, 1:2]
    h2 = (jnp.dot(s2_ref[0, :N, :] * invc, wl_ref[:128, :],
                  preferred_element_type=jnp.float32)
          + jnp.dot(s2_ref[1, :N, :] * invc, wl_ref[128:, :],
                    preferred_element_type=jnp.float32)
          + jnp.dot(h1_ref[0, :N, :], wr_ref[:128, :],
                    preferred_element_type=jnp.float32)
          + jnp.dot(h1_ref[1, :N, :], wr_ref[128:, :],
                    preferred_element_type=jnp.float32)
          + bs_ref[...][None, :])
    h2 = _bn_leaky(h2, g2_ref[...], be2_ref[...])
    y3 = jnp.dot(h2, w3_ref[...], preferred_element_type=jnp.float32)
    y3_ref[:N, :64] = y3 * dv_ref[:N, 0:1]
    y3_ref[:N, 64:] = jnp.zeros((N, 64), jnp.float32)
    y3_ref[N:, :] = jnp.zeros((N_ACC - N, 128), jnp.float32)


def _tc_d(s3_ref, y3_ref, dv_ref, b3_ref, g3_ref, be3_ref, w4_ref, y4_ref):
    dinv = dv_ref[:N, 0:1]
    t = dinv * (s3_ref[0, :N, :64] + s3_ref[1, :N, :64]
                + y3_ref[:N, :64]) + b3_ref[...][None, :]
    h3 = _bn_leaky(t, g3_ref[...], be3_ref[...])
    y4 = jnp.dot(h3, w4_ref[...], preferred_element_type=jnp.float32)
    y4_ref[:N, :64] = y4 * dinv
    y4_ref[:N, 64:] = jnp.zeros((N, 64), jnp.float32)
    y4_ref[N:, :] = jnp.zeros((N_ACC - N, 128), jnp.float32)


def _tc_e(s4_ref, y4_ref, dv_ref, b4_ref, h4_ref):
    dinv = dv_ref[:N, 0:1]
    h4_ref[:N, :] = dinv * (s4_ref[0, :N, :64] + s4_ref[1, :N, :64]
                            + y4_ref[:N, :64]) + b4_ref[...][None, :]
    h4_ref[N:, :] = jnp.zeros((N_POOL - N, 64), jnp.float32)


def _tc_f(ga_ref, gb_ref, out_ref):
    se = ga_ref[:, :64] + ga_ref[:, 64:]
    te = gb_ref[:, :64] + gb_ref[:, 64:]
    d = jnp.sum(se * te, axis=1)
    out_ref[...] = 1.0 / (1.0 + jnp.exp(-d))


def _tc(fn, out_shape, *args):
    return pl.pallas_call(fn, out_shape=out_shape)(*args)


def kernel(x, edge_index, batch, link_indices, W1, b1, g1, be1, Wl, Wr, bs,
           g2, be2, W3, b3, g3, be3, W4, b4):
    f32 = jnp.float32
    src = edge_index[0]
    dst = edge_index[1]
    pad = E_PAD - E
    srca = jnp.concatenate([src, jnp.zeros((pad,), jnp.int32)])
    srca = srca.reshape(NCHUNK, CH)
    srcb = srca + N_ACC
    dstp = jnp.concatenate([dst, jnp.full((pad,), N, jnp.int32)])
    dstp = dstp.reshape(NCHUNK, CH)
    batch2 = jnp.concatenate([batch, jnp.full((N_POOL - N,), G, jnp.int32)])
    batch2 = batch2.reshape(NW, N_POOL // NW)
    batch2 = jnp.pad(batch2, ((0, 0), (0, 8 * CH - N_POOL // NW)),
                     constant_values=G)
    batch2 = batch2.reshape(NW * 8, CH)
    li = jnp.concatenate([link_indices[0], link_indices[1]])

    deg2 = _sc_deg(dstp)
    y1, dvs = _tc(_tc_a,
                  (jax.ShapeDtypeStruct((NC, N_ACC, 128), f32),
                   jax.ShapeDtypeStruct((N_ACC, 2), f32)),
                  x, W1, deg2)
    s1 = _sc_agg256(y1.reshape(NC * N_ACC, 128), srca, srcb, dstp)
    h1 = _tc(_tc_b, jax.ShapeDtypeStruct((NC, N_ACC, 128), f32),
             s1, y1, dvs, b1, g1, be1)
    s2 = _sc_agg256(h1.reshape(NC * N_ACC, 128), srca, srcb, dstp)
    y3 = _tc(_tc_c, jax.ShapeDtypeStruct((N_ACC, 128), f32),
             s2, h1, dvs, Wl, Wr, bs, g2, be2, W3)
    s3 = _sc_agg64(y3, srca, srcb, dstp)
    y4 = _tc(_tc_d, jax.ShapeDtypeStruct((N_ACC, 128), f32),
             s3, y3, dvs, b3, g3, be3, W4)
    s4 = _sc_agg64(y4, srca, srcb, dstp)
    h4 = _tc(_tc_e, jax.ShapeDtypeStruct((N_POOL, 64), f32),
             s4, y4, dvs, b4)
    pool2 = _sc_pool(h4, batch2)
    pcat = jnp.concatenate([pool2[0], pool2[1]], axis=1)
    ga, gb = _sc_links(pcat, li)
    return _tc(_tc_f, jax.ShapeDtypeStruct((NLINK,), f32), ga, gb)


# speedup vs baseline: 19.4670x; 2.8512x over previous
import functools

import jax
import jax.numpy as jnp
from jax import lax
from jax.experimental import pallas as pl
from jax.experimental.pallas import tpu as pltpu
from jax.experimental.pallas import tpu_sc as plsc

N = 10000
E = 320000
G = 1024
NLINK = 2048
NC = 2
NS = 16
NW = NC * NS
CH = 128
BLK = 40
NCHUNK = 2560
E_PAD = NCHUNK * CH
N_ACC = 10112
POOL_R = 1152
N_POOL = 12288

_mesh = lambda: plsc.VectorSubcoreMesh(
    core_axis_name="c", subcore_axis_name="s", num_cores=NC, num_subcores=NS)


@functools.partial(
    pl.kernel,
    out_type=jax.ShapeDtypeStruct((NC, N_ACC, 16), jnp.float32),
    mesh=_mesh(),
    scratch_types=[
        pltpu.VMEM((NCHUNK // NW, CH), jnp.int32),
        pltpu.VMEM((CH, 16), jnp.float32),
        pltpu.VMEM((CH, 16), jnp.float32),
        pltpu.VMEM_SHARED((N_ACC, 16), jnp.float32),
    ],
)
def _sc_deg(dst_hbm, out_hbm, idst, onesb, zerob, spm):
    c = lax.axis_index("c")
    s = lax.axis_index("s")
    w = c * NS + s
    rpt = NCHUNK // NW
    zero16 = jnp.zeros((16,), jnp.float32)
    ones16 = jnp.ones((16,), jnp.float32)

    def fill(i, _):
        onesb[i, pl.ds(0, 16)] = ones16
        zerob[i, pl.ds(0, 16)] = zero16
        return 0
    lax.fori_loop(0, CH, fill, 0)
    pltpu.sync_copy(dst_hbm.at[pl.ds(w * rpt, rpt)], idst)
    wb = N_ACC // NS
    zb = s * wb
    for k in range(wb // CH):
        pltpu.sync_copy(zerob, spm.at[pl.ds(zb + k * CH, CH)])
    pltpu.sync_copy(zerob.at[pl.ds(0, wb % CH)],
                    spm.at[pl.ds(zb + (wb // CH) * CH, wb % CH)])
    plsc.subcore_barrier()

    def acc(j, _):
        pltpu.sync_copy(onesb, spm.at[idst.at[j]], add=True)
        return 0
    lax.fori_loop(0, rpt, acc, 0)
    plsc.subcore_barrier()
    pltpu.sync_copy(spm.at[pl.ds(s * wb, wb)], out_hbm.at[c, pl.ds(s * wb, wb)])


def _make_sc_agg(feature_split):
    ptc = NCHUNK // NS if feature_split else NCHUNK // NW
    nblk = ptc // BLK
    wb = N_ACC // NS

    @functools.partial(
        pl.kernel,
        out_type=jax.ShapeDtypeStruct((NC, N_ACC, 128), jnp.float32),
        mesh=_mesh(),
        scratch_types=[
            pltpu.VMEM((BLK, CH), jnp.int32),
            pltpu.VMEM((BLK, CH), jnp.int32),
            pltpu.VMEM((CH, 128), jnp.float32),
            pltpu.VMEM((CH, 128), jnp.float32),
            pltpu.VMEM_SHARED((N_ACC, 128), jnp.float32),
            pltpu.SemaphoreType.DMA,
            pltpu.SemaphoreType.DMA,
        ],
    )
    def agg(h_hbm, srca_hbm, srcb_hbm, dst_hbm, out_hbm,
            isrc, idst, buf0, buf1, acc, gs0, gs1):
        bufs = (buf0, buf1)
        gsem = (gs0, gs1)
        c = lax.axis_index("c")
        s = lax.axis_index("s")
        if feature_split:
            base = s * ptc
        else:
            base = (c * NS + s) * ptc
        zero16 = jnp.zeros((16,), jnp.float32)

        def zloop(i, _):
            buf0[i >> 3, pl.ds((i & 7) * 16, 16)] = zero16
            return 0
        lax.fori_loop(0, CH * 8, zloop, 0)
        zb = s * wb
        for k in range(wb // CH):
            pltpu.sync_copy(buf0, acc.at[pl.ds(zb + k * CH, CH)])
        pltpu.sync_copy(buf0.at[pl.ds(0, wb % CH)],
                        acc.at[pl.ds(zb + (wb // CH) * CH, wb % CH)])
        plsc.subcore_barrier()

        def blk(bi, _):
            bb = base + bi * BLK
            if feature_split:
                @pl.when(c == 0)
                def _():
                    pltpu.sync_copy(srca_hbm.at[pl.ds(bb, BLK)], isrc)

                @pl.when(c == 1)
                def _():
                    pltpu.sync_copy(srcb_hbm.at[pl.ds(bb, BLK)], isrc)
            else:
                pltpu.sync_copy(srca_hbm.at[pl.ds(bb, BLK)], isrc)
            pltpu.sync_copy(dst_hbm.at[pl.ds(bb, BLK)], idst)
            pltpu.async_copy(h_hbm.at[isrc.at[0]], buf0, gs0)
            pltpu.async_copy(h_hbm.at[isrc.at[1]], buf1, gs1)

            def step(k, _):
                for p in (0, 1):
                    @pl.when(k % 2 == p)
                    def _(p=p):
                        pltpu.make_async_copy(
                            h_hbm.at[isrc.at[k]], bufs[p], gsem[p]).wait()
                        pltpu.sync_copy(bufs[p], acc.at[idst.at[k]], add=True)

                        @pl.when(k + 2 < BLK)
                        def _():
                            pltpu.async_copy(h_hbm.at[isrc.at[k + 2]],
                                             bufs[p], gsem[p])
                return 0
            lax.fori_loop(0, BLK, step, 0)
            return 0
        lax.fori_loop(0, nblk, blk, 0)
        plsc.subcore_barrier()
        pltpu.sync_copy(acc.at[pl.ds(s * wb, wb)],
                        out_hbm.at[c, pl.ds(s * wb, wb)])

    return agg


_sc_agg256 = _make_sc_agg(True)
_sc_agg64 = _make_sc_agg(False)


@functools.partial(
    pl.kernel,
    out_type=jax.ShapeDtypeStruct((NC, POOL_R, 64), jnp.float32),
    mesh=_mesh(),
    scratch_types=[
        pltpu.VMEM((N_POOL // NW, 64), jnp.float32),
        pltpu.VMEM((8, CH), jnp.int32),
        pltpu.VMEM((CH, 64), jnp.float32),
        pltpu.VMEM_SHARED((POOL_R, 64), jnp.float32),
    ],
)
def _sc_pool(h_hbm, batch_hbm, out_hbm, rows, bidx, zerob, psh):
    c = lax.axis_index("c")
    s = lax.axis_index("s")
    w = c * NS + s
    npt = N_POOL // NW
    zero16 = jnp.zeros((16,), jnp.float32)

    def zloop(i, _):
        zerob[i >> 2, pl.ds((i & 3) * 16, 16)] = zero16
        return 0
    lax.fori_loop(0, CH * 4, zloop, 0)
    pltpu.sync_copy(h_hbm.at[pl.ds(w * npt, npt)], rows)
    pltpu.sync_copy(batch_hbm.at[pl.ds(w * 8, 8)], bidx)
    zr = POOL_R // NS
    pltpu.sync_copy(zerob.at[pl.ds(0, zr)], psh.at[pl.ds(s * zr, zr)])
    plsc.subcore_barrier()
    for k in range(npt // CH):
        pltpu.sync_copy(rows.at[pl.ds(k * CH, CH)], psh.at[bidx.at[k]],
                        add=True)
    plsc.subcore_barrier()
    pltpu.sync_copy(psh.at[pl.ds(s * zr, zr)], out_hbm.at[c, pl.ds(s * zr, zr)])


@functools.partial(
    pl.kernel,
    out_type=[jax.ShapeDtypeStruct((NLINK, 128), jnp.float32),
              jax.ShapeDtypeStruct((NLINK, 128), jnp.float32)],
    mesh=_mesh(),
    scratch_types=[
        pltpu.VMEM((64,), jnp.int32),
        pltpu.VMEM((64,), jnp.int32),
        pltpu.VMEM((64, 128), jnp.float32),
        pltpu.VMEM((64, 128), jnp.float32),
        pltpu.SemaphoreType.DMA,
    ],
)
def _sc_links(pool_hbm, li_hbm, ga_hbm, gb_hbm, ia, ib, ra, rb, sem):
    c = lax.axis_index("c")
    s = lax.axis_index("s")
    w = c * NS + s
    lpt = NLINK // NW
    pltpu.sync_copy(li_hbm.at[pl.ds(w * lpt, lpt)], ia)
    pltpu.sync_copy(li_hbm.at[pl.ds(NLINK + w * lpt, lpt)], ib)
    pltpu.async_copy(pool_hbm.at[ia], ra, sem).wait()
    pltpu.async_copy(pool_hbm.at[ib], rb, sem).wait()
    pltpu.sync_copy(ra, ga_hbm.at[pl.ds(w * lpt, lpt)])
    pltpu.sync_copy(rb, gb_hbm.at[pl.ds(w * lpt, lpt)])


def _bn_leaky(t, g, be):
    mu = jnp.mean(t, axis=0, keepdims=True)
    var = jnp.mean((t - mu) ** 2, axis=0, keepdims=True)
    bn = (t - mu) * lax.rsqrt(var + 1e-5) * g[None, :] + be[None, :]
    return jnp.where(bn >= 0, bn, 0.01 * bn)


def _tc_a(x_ref, w1_ref, deg_ref, y1_ref, dv_ref):
    deg = (deg_ref[0] + deg_ref[1])[:N, 0:1]
    dinv = lax.rsqrt(deg + 1.0)
    invc = 1.0 / jnp.maximum(deg, 1.0)
    xw = jnp.dot(x_ref[...], w1_ref[...], preferred_element_type=jnp.float32)
    y = xw * dinv
    y1_ref[0, :N, :] = y[:, :128]
    y1_ref[1, :N, :] = y[:, 128:]
    y1_ref[0, N:, :] = jnp.zeros((N_ACC - N, 128), jnp.float32)
    y1_ref[1, N:, :] = jnp.zeros((N_ACC - N, 128), jnp.float32)
    dv_ref[:N, :] = jnp.concatenate([dinv, invc], axis=1)
    dv_ref[N:, :] = jnp.zeros((N_ACC - N, 2), jnp.float32)


def _tc_b(s1_ref, y1_ref, dv_ref, b1_ref, g1_ref, be1_ref, h1_ref):
    dinv = dv_ref[:N, 0:1]
    for h in (0, 1):
        sl = slice(h * 128, (h + 1) * 128)
        t = dinv * (s1_ref[h, :N, :] + y1_ref[h, :N, :]) + b1_ref[sl][None, :]
        h1_ref[h, :N, :] = _bn_leaky(t, g1_ref[sl], be1_ref[sl])
        h1_ref[h, N:, :] = jnp.zeros((N_ACC - N, 128), jnp.float32)


def _tc_c(s2_ref, h1_ref, dv_ref, wl_ref, wr_ref, bs_ref, g2_ref, be2_ref,
          w3_ref, y3_ref):
    invc = dv_ref[:N, 1:2]
    h2 = (jnp.dot(s2_ref[0, :N, :] * invc, wl_ref[:128, :],
                  preferred_element_type=jnp.float32)
          + jnp.dot(s2_ref[1, :N, :] * invc, wl_ref[128:, :],
                    preferred_element_type=jnp.float32)
          + jnp.dot(h1_ref[0, :N, :], wr_ref[:128, :],
                    preferred_element_type=jnp.float32)
          + jnp.dot(h1_ref[1, :N, :], wr_ref[128:, :],
                    preferred_element_type=jnp.float32)
          + bs_ref[...][None, :])
    h2 = _bn_leaky(h2, g2_ref[...], be2_ref[...])
    y3 = jnp.dot(h2, w3_ref[...], preferred_element_type=jnp.float32)
    y3_ref[:N, :64] = y3 * dv_ref[:N, 0:1]
    y3_ref[:N, 64:] = jnp.zeros((N, 64), jnp.float32)
    y3_ref[N:, :] = jnp.zeros((N_ACC - N, 128), jnp.float32)


def _tc_d(s3_ref, y3_ref, dv_ref, b3_ref, g3_ref, be3_ref, w4_ref, y4_ref):
    dinv = dv_ref[:N, 0:1]
    t = dinv * (s3_ref[0, :N, :64] + s3_ref[1, :N, :64]
                + y3_ref[:N, :64]) + b3_ref[...][None, :]
    h3 = _bn_leaky(t, g3_ref[...], be3_ref[...])
    y4 = jnp.dot(h3, w4_ref[...], preferred_element_type=jnp.float32)
    y4_ref[:N, :64] = y4 * dinv
    y4_ref[:N, 64:] = jnp.zeros((N, 64), jnp.float32)
    y4_ref[N:, :] = jnp.zeros((N_ACC - N, 128), jnp.float32)


def _tc_e(s4_ref, y4_ref, dv_ref, b4_ref, h4_ref):
    dinv = dv_ref[:N, 0:1]
    h4_ref[:N, :] = dinv * (s4_ref[0, :N, :64] + s4_ref[1, :N, :64]
                            + y4_ref[:N, :64]) + b4_ref[...][None, :]
    h4_ref[N:, :] = jnp.zeros((N_POOL - N, 64), jnp.float32)


def _tc_f(ga_ref, gb_ref, out_ref):
    se = ga_ref[:, :64] + ga_ref[:, 64:]
    te = gb_ref[:, :64] + gb_ref[:, 64:]
    d = jnp.sum(se * te, axis=1)
    out_ref[...] = 1.0 / (1.0 + jnp.exp(-d))


def _tc(fn, out_shape, *args):
    return pl.pallas_call(fn, out_shape=out_shape)(*args)


def kernel(x, edge_index, batch, link_indices, W1, b1, g1, be1, Wl, Wr, bs,
           g2, be2, W3, b3, g3, be3, W4, b4):
    f32 = jnp.float32
    src = edge_index[0]
    dst = edge_index[1]
    pad = E_PAD - E
    spread = jnp.arange(pad, dtype=jnp.int32)
    srca = jnp.concatenate([src, spread % N])
    srca = srca.reshape(NCHUNK, CH)
    srcb = srca + N_ACC
    dstp = jnp.concatenate([dst, N + spread % (N_ACC - N)])
    dstp = dstp.reshape(NCHUNK, CH)
    batch2 = jnp.concatenate([batch, jnp.full((N_POOL - N,), G, jnp.int32)])
    batch2 = batch2.reshape(NW, N_POOL // NW)
    batch2 = jnp.pad(batch2, ((0, 0), (0, 8 * CH - N_POOL // NW)),
                     constant_values=G)
    batch2 = batch2.reshape(NW * 8, CH)
    li = jnp.concatenate([link_indices[0], link_indices[1]])

    deg2 = _sc_deg(dstp)
    y1, dvs = _tc(_tc_a,
                  (jax.ShapeDtypeStruct((NC, N_ACC, 128), f32),
                   jax.ShapeDtypeStruct((N_ACC, 2), f32)),
                  x, W1, deg2)
    s1 = _sc_agg256(y1.reshape(NC * N_ACC, 128), srca, srcb, dstp)
    h1 = _tc(_tc_b, jax.ShapeDtypeStruct((NC, N_ACC, 128), f32),
             s1, y1, dvs, b1, g1, be1)
    s2 = _sc_agg256(h1.reshape(NC * N_ACC, 128), srca, srcb, dstp)
    y3 = _tc(_tc_c, jax.ShapeDtypeStruct((N_ACC, 128), f32),
             s2, h1, dvs, Wl, Wr, bs, g2, be2, W3)
    s3 = _sc_agg64(y3, srca, srcb, dstp)
    y4 = _tc(_tc_d, jax.ShapeDtypeStruct((N_ACC, 128), f32),
             s3, y3, dvs, b3, g3, be3, W4)
    s4 = _sc_agg64(y4, srca, srcb, dstp)
    h4 = _tc(_tc_e, jax.ShapeDtypeStruct((N_POOL, 64), f32),
             s4, y4, dvs, b4)
    pool2 = _sc_pool(h4, batch2)
    pcat = jnp.concatenate([pool2[0], pool2[1]], axis=1)
    ga, gb = _sc_links(pcat, li)
    return _tc(_tc_f, jax.ShapeDtypeStruct((NLINK,), f32), ga, gb)


# speedup vs baseline: 20.9838x; 1.0779x over previous
import functools

import jax
import jax.numpy as jnp
from jax import lax
from jax.experimental import pallas as pl
from jax.experimental.pallas import tpu as pltpu
from jax.experimental.pallas import tpu_sc as plsc

N = 10000
E = 320000
G = 1024
NLINK = 2048
NC = 2
NS = 16
NW = NC * NS
CH = 128
BLK = 40
NCHUNK = 2560
E_PAD = NCHUNK * CH
N_ACC = 10112
POOL_R = 1152
N_POOL = 12288

_mesh = lambda: plsc.VectorSubcoreMesh(
    core_axis_name="c", subcore_axis_name="s", num_cores=NC, num_subcores=NS)


@functools.partial(
    pl.kernel,
    out_type=jax.ShapeDtypeStruct((NC, N_ACC, 16), jnp.float32),
    mesh=_mesh(),
    scratch_types=[
        pltpu.VMEM((NCHUNK // NW, CH), jnp.int32),
        pltpu.VMEM((CH, 16), jnp.float32),
        pltpu.VMEM((CH, 16), jnp.float32),
        pltpu.VMEM_SHARED((N_ACC, 16), jnp.float32),
    ],
)
def _sc_deg(dst_hbm, out_hbm, idst, onesb, zerob, spm):
    c = lax.axis_index("c")
    s = lax.axis_index("s")
    w = c * NS + s
    rpt = NCHUNK // NW
    zero16 = jnp.zeros((16,), jnp.float32)
    ones16 = jnp.ones((16,), jnp.float32)

    def fill(i, _):
        onesb[i, pl.ds(0, 16)] = ones16
        zerob[i, pl.ds(0, 16)] = zero16
        return 0
    lax.fori_loop(0, CH, fill, 0)
    pltpu.sync_copy(dst_hbm.at[pl.ds(w * rpt, rpt)], idst)
    wb = N_ACC // NS
    zb = s * wb
    for k in range(wb // CH):
        pltpu.sync_copy(zerob, spm.at[pl.ds(zb + k * CH, CH)])
    pltpu.sync_copy(zerob.at[pl.ds(0, wb % CH)],
                    spm.at[pl.ds(zb + (wb // CH) * CH, wb % CH)])
    plsc.subcore_barrier()

    def acc(j, _):
        pltpu.sync_copy(onesb, spm.at[idst.at[j]], add=True)
        return 0
    lax.fori_loop(0, rpt, acc, 0)
    plsc.subcore_barrier()
    pltpu.sync_copy(spm.at[pl.ds(s * wb, wb)], out_hbm.at[c, pl.ds(s * wb, wb)])


def _make_sc_agg(F, feature_split):
    ptc = NCHUNK // NS if feature_split else NCHUNK // NW
    nblk = ptc // BLK
    wb = N_ACC // NS

    @functools.partial(
        pl.kernel,
        out_type=jax.ShapeDtypeStruct((NC, N_ACC, F), jnp.float32),
        mesh=_mesh(),
        compiler_params=pltpu.CompilerParams(
            use_tc_tiling_on_sc=(F == 128)),
        scratch_types=[
            pltpu.VMEM((BLK, CH), jnp.int32),
            pltpu.VMEM((BLK, CH), jnp.int32),
            pltpu.VMEM((CH, F), jnp.float32),
            pltpu.VMEM((CH, F), jnp.float32),
            pltpu.VMEM_SHARED((N_ACC, F), jnp.float32),
            pltpu.SemaphoreType.DMA,
            pltpu.SemaphoreType.DMA,
        ],
    )
    def agg(h_hbm, srca_hbm, srcb_hbm, dst_hbm, out_hbm,
            isrc, idst, buf0, buf1, acc, gs0, gs1):
        bufs = (buf0, buf1)
        gsem = (gs0, gs1)
        c = lax.axis_index("c")
        s = lax.axis_index("s")
        if feature_split:
            base = s * ptc
        else:
            base = (c * NS + s) * ptc
        zero16 = jnp.zeros((16,), jnp.float32)

        nv = F // 16

        def zloop(i, _):
            buf0[i // nv, pl.ds((i % nv) * 16, 16)] = zero16
            return 0
        lax.fori_loop(0, CH * nv, zloop, 0)
        zb = s * wb
        for k in range(wb // CH):
            pltpu.sync_copy(buf0, acc.at[pl.ds(zb + k * CH, CH)])
        pltpu.sync_copy(buf0.at[pl.ds(0, wb % CH)],
                        acc.at[pl.ds(zb + (wb // CH) * CH, wb % CH)])
        plsc.subcore_barrier()

        def blk(bi, _):
            bb = base + bi * BLK
            if feature_split:
                @pl.when(c == 0)
                def _():
                    pltpu.sync_copy(srca_hbm.at[pl.ds(bb, BLK)], isrc)

                @pl.when(c == 1)
                def _():
                    pltpu.sync_copy(srcb_hbm.at[pl.ds(bb, BLK)], isrc)
            else:
                pltpu.sync_copy(srca_hbm.at[pl.ds(bb, BLK)], isrc)
            pltpu.sync_copy(dst_hbm.at[pl.ds(bb, BLK)], idst)
            pltpu.async_copy(h_hbm.at[isrc.at[0]], buf0, gs0)
            pltpu.async_copy(h_hbm.at[isrc.at[1]], buf1, gs1)

            def step(k, _):
                for p in (0, 1):
                    @pl.when(k % 2 == p)
                    def _(p=p):
                        pltpu.make_async_copy(
                            h_hbm.at[isrc.at[k]], bufs[p], gsem[p]).wait()
                        pltpu.sync_copy(bufs[p], acc.at[idst.at[k]], add=True)

                        @pl.when(k + 2 < BLK)
                        def _():
                            pltpu.async_copy(h_hbm.at[isrc.at[k + 2]],
                                             bufs[p], gsem[p])
                return 0
            lax.fori_loop(0, BLK, step, 0)
            return 0
        lax.fori_loop(0, nblk, blk, 0)
        plsc.subcore_barrier()
        pltpu.sync_copy(acc.at[pl.ds(s * wb, wb)],
                        out_hbm.at[c, pl.ds(s * wb, wb)])

    return agg


_sc_agg256 = _make_sc_agg(128, True)
_sc_agg64 = _make_sc_agg(64, False)


@functools.partial(
    pl.kernel,
    out_type=jax.ShapeDtypeStruct((NC, POOL_R, 64), jnp.float32),
    mesh=_mesh(),
    scratch_types=[
        pltpu.VMEM((N_POOL // NW, 64), jnp.float32),
        pltpu.VMEM((8, CH), jnp.int32),
        pltpu.VMEM((CH, 64), jnp.float32),
        pltpu.VMEM_SHARED((POOL_R, 64), jnp.float32),
    ],
)
def _sc_pool(h_hbm, batch_hbm, out_hbm, rows, bidx, zerob, psh):
    c = lax.axis_index("c")
    s = lax.axis_index("s")
    w = c * NS + s
    npt = N_POOL // NW
    zero16 = jnp.zeros((16,), jnp.float32)

    def zloop(i, _):
        zerob[i >> 2, pl.ds((i & 3) * 16, 16)] = zero16
        return 0
    lax.fori_loop(0, CH * 4, zloop, 0)
    pltpu.sync_copy(h_hbm.at[pl.ds(w * npt, npt)], rows)
    pltpu.sync_copy(batch_hbm.at[pl.ds(w * 8, 8)], bidx)
    zr = POOL_R // NS
    pltpu.sync_copy(zerob.at[pl.ds(0, zr)], psh.at[pl.ds(s * zr, zr)])
    plsc.subcore_barrier()
    for k in range(npt // CH):
        pltpu.sync_copy(rows.at[pl.ds(k * CH, CH)], psh.at[bidx.at[k]],
                        add=True)
    plsc.subcore_barrier()
    pltpu.sync_copy(psh.at[pl.ds(s * zr, zr)], out_hbm.at[c, pl.ds(s * zr, zr)])


@functools.partial(
    pl.kernel,
    out_type=[jax.ShapeDtypeStruct((NLINK, 128), jnp.float32),
              jax.ShapeDtypeStruct((NLINK, 128), jnp.float32)],
    mesh=_mesh(),
    scratch_types=[
        pltpu.VMEM((64,), jnp.int32),
        pltpu.VMEM((64,), jnp.int32),
        pltpu.VMEM((64, 128), jnp.float32),
        pltpu.VMEM((64, 128), jnp.float32),
        pltpu.SemaphoreType.DMA,
    ],
)
def _sc_links(pool_hbm, li_hbm, ga_hbm, gb_hbm, ia, ib, ra, rb, sem):
    c = lax.axis_index("c")
    s = lax.axis_index("s")
    w = c * NS + s
    lpt = NLINK // NW
    pltpu.sync_copy(li_hbm.at[pl.ds(w * lpt, lpt)], ia)
    pltpu.sync_copy(li_hbm.at[pl.ds(NLINK + w * lpt, lpt)], ib)
    pltpu.async_copy(pool_hbm.at[ia], ra, sem).wait()
    pltpu.async_copy(pool_hbm.at[ib], rb, sem).wait()
    pltpu.sync_copy(ra, ga_hbm.at[pl.ds(w * lpt, lpt)])
    pltpu.sync_copy(rb, gb_hbm.at[pl.ds(w * lpt, lpt)])


def _bn_leaky(t, g, be):
    mu = jnp.mean(t, axis=0, keepdims=True)
    var = jnp.mean((t - mu) ** 2, axis=0, keepdims=True)
    bn = (t - mu) * lax.rsqrt(var + 1e-5) * g[None, :] + be[None, :]
    return jnp.where(bn >= 0, bn, 0.01 * bn)


def _tc_a(x_ref, w1_ref, deg_ref, y1_ref, dv_ref):
    deg = (deg_ref[0] + deg_ref[1])[:N, 0:1]
    dinv = lax.rsqrt(deg + 1.0)
    invc = 1.0 / jnp.maximum(deg, 1.0)
    xw = jnp.dot(x_ref[...], w1_ref[...], preferred_element_type=jnp.float32)
    y = xw * dinv
    y1_ref[0, :N, :] = y[:, :128]
    y1_ref[1, :N, :] = y[:, 128:]
    y1_ref[0, N:, :] = jnp.zeros((N_ACC - N, 128), jnp.float32)
    y1_ref[1, N:, :] = jnp.zeros((N_ACC - N, 128), jnp.float32)
    dv_ref[:N, :] = jnp.concatenate([dinv, invc], axis=1)
    dv_ref[N:, :] = jnp.zeros((N_ACC - N, 2), jnp.float32)


def _tc_b(s1_ref, y1_ref, dv_ref, b1_ref, g1_ref, be1_ref, h1_ref):
    dinv = dv_ref[:N, 0:1]
    for h in (0, 1):
        sl = slice(h * 128, (h + 1) * 128)
        t = dinv * (s1_ref[h, :N, :] + y1_ref[h, :N, :]) + b1_ref[sl][None, :]
        h1_ref[h, :N, :] = _bn_leaky(t, g1_ref[sl], be1_ref[sl])
        h1_ref[h, N:, :] = jnp.zeros((N_ACC - N, 128), jnp.float32)


def _tc_c(s2_ref, h1_ref, dv_ref, wl_ref, wr_ref, bs_ref, g2_ref, be2_ref,
          w3_ref, y3_ref):
    invc = dv_ref[:N, 1:2]
    h2 = (jnp.dot(s2_ref[0, :N, :] * invc, wl_ref[:128, :],
                  preferred_element_type=jnp.float32)
          + jnp.dot(s2_ref[1, :N, :] * invc, wl_ref[128:, :],
                    preferred_element_type=jnp.float32)
          + jnp.dot(h1_ref[0, :N, :], wr_ref[:128, :],
                    preferred_element_type=jnp.float32)
          + jnp.dot(h1_ref[1, :N, :], wr_ref[128:, :],
                    preferred_element_type=jnp.float32)
          + bs_ref[...][None, :])
    h2 = _bn_leaky(h2, g2_ref[...], be2_ref[...])
    y3 = jnp.dot(h2, w3_ref[...], preferred_element_type=jnp.float32)
    y3_ref[:N, :] = y3 * dv_ref[:N, 0:1]
    y3_ref[N:, :] = jnp.zeros((N_ACC - N, 64), jnp.float32)


def _tc_d(s3_ref, y3_ref, dv_ref, b3_ref, g3_ref, be3_ref, w4_ref, y4_ref):
    dinv = dv_ref[:N, 0:1]
    t = dinv * (s3_ref[0, :N, :] + s3_ref[1, :N, :]
                + y3_ref[:N, :]) + b3_ref[...][None, :]
    h3 = _bn_leaky(t, g3_ref[...], be3_ref[...])
    y4 = jnp.dot(h3, w4_ref[...], preferred_element_type=jnp.float32)
    y4_ref[:N, :] = y4 * dinv
    y4_ref[N:, :] = jnp.zeros((N_ACC - N, 64), jnp.float32)


def _tc_e(s4_ref, y4_ref, dv_ref, b4_ref, h4_ref):
    dinv = dv_ref[:N, 0:1]
    h4_ref[:N, :] = dinv * (s4_ref[0, :N, :] + s4_ref[1, :N, :]
                            + y4_ref[:N, :]) + b4_ref[...][None, :]
    h4_ref[N:, :] = jnp.zeros((N_POOL - N, 64), jnp.float32)


def _tc_f(ga_ref, gb_ref, out_ref):
    se = ga_ref[:, :64] + ga_ref[:, 64:]
    te = gb_ref[:, :64] + gb_ref[:, 64:]
    d = jnp.sum(se * te, axis=1)
    out_ref[...] = 1.0 / (1.0 + jnp.exp(-d))


def _tc(fn, out_shape, *args):
    return pl.pallas_call(fn, out_shape=out_shape)(*args)


def kernel(x, edge_index, batch, link_indices, W1, b1, g1, be1, Wl, Wr, bs,
           g2, be2, W3, b3, g3, be3, W4, b4):
    f32 = jnp.float32
    src = edge_index[0]
    dst = edge_index[1]
    pad = E_PAD - E
    spread = jnp.arange(pad, dtype=jnp.int32)
    srca = jnp.concatenate([src, spread % N])
    srca = srca.reshape(NCHUNK, CH)
    srcb = srca + N_ACC
    dstp = jnp.concatenate([dst, N + spread % (N_ACC - N)])
    dstp = dstp.reshape(NCHUNK, CH)
    batch2 = jnp.concatenate([batch, jnp.full((N_POOL - N,), G, jnp.int32)])
    batch2 = batch2.reshape(NW, N_POOL // NW)
    batch2 = jnp.pad(batch2, ((0, 0), (0, 8 * CH - N_POOL // NW)),
                     constant_values=G)
    batch2 = batch2.reshape(NW * 8, CH)
    li = jnp.concatenate([link_indices[0], link_indices[1]])

    deg2 = _sc_deg(dstp)
    y1, dvs = _tc(_tc_a,
                  (jax.ShapeDtypeStruct((NC, N_ACC, 128), f32),
                   jax.ShapeDtypeStruct((N_ACC, 2), f32)),
                  x, W1, deg2)
    s1 = _sc_agg256(y1.reshape(NC * N_ACC, 128), srca, srcb, dstp)
    h1 = _tc(_tc_b, jax.ShapeDtypeStruct((NC, N_ACC, 128), f32),
             s1, y1, dvs, b1, g1, be1)
    s2 = _sc_agg256(h1.reshape(NC * N_ACC, 128), srca, srcb, dstp)
    y3 = _tc(_tc_c, jax.ShapeDtypeStruct((N_ACC, 64), f32),
             s2, h1, dvs, Wl, Wr, bs, g2, be2, W3)
    s3 = _sc_agg64(y3, srca, srcb, dstp)
    y4 = _tc(_tc_d, jax.ShapeDtypeStruct((N_ACC, 64), f32),
             s3, y3, dvs, b3, g3, be3, W4)
    s4 = _sc_agg64(y4, srca, srcb, dstp)
    h4 = _tc(_tc_e, jax.ShapeDtypeStruct((N_POOL, 64), f32),
             s4, y4, dvs, b4)
    pool2 = _sc_pool(h4, batch2)
    pcat = jnp.concatenate([pool2[0], pool2[1]], axis=1)
    ga, gb = _sc_links(pcat, li)
    return _tc(_tc_f, jax.ShapeDtypeStruct((NLINK,), f32), ga, gb)
